# baseline (speedup 1.0000x reference)
import jax
import jax.numpy as jnp
from jax import lax
from jax.experimental import pallas as pl
from jax.experimental.pallas import tpu as pltpu
from jax.experimental.pallas import tpu_sc as plsc

_NC, _NS, _L = 2, 16, 16
_NW = _NC * _NS
_BATCH = 16384
_BPW = _BATCH // _NW
_CHUNK = 128
_NCH = _BPW // _CHUNK
_D = 32
_NB = 1000
_QPERM = [v for k in range(16) for v in (k, 16 + k)]


def _tec_body(uid_hbm, ts_hbm, utab_hbm, ttab_hbm, bkt_hbm, cst_hbm,
              out_hbm,
              uid_v, ts_v, bidx_v, urows_v, trows_v, bkt_v, cst_v, norm_v,
              outbuf_v, sem_in, sem_g, sem_out):
    wid = lax.axis_index("s") * _NC + lax.axis_index("c")
    base = wid * _BPW

    stage = [
        pltpu.async_copy(uid_hbm.at[wid], uid_v, sem_in),
        pltpu.async_copy(ts_hbm.at[pl.ds(base, _BPW)], ts_v, sem_in),
        pltpu.async_copy(bkt_hbm, bkt_v, sem_in),
        pltpu.async_copy(cst_hbm, cst_v, sem_in),
    ]
    for cp in stage:
        cp.wait()

    ucopies = [
        pltpu.async_copy(utab_hbm.at[uid_v.at[c]],
                         urows_v.at[pl.ds(c * _CHUNK, _CHUNK)], sem_g)
        for c in range(_NCH)
    ]

    mean = cst_v[0]
    inv = cst_v[1]
    lanes = lax.iota(jnp.int32, _L)
    zero = jnp.zeros((_L,), jnp.int32)
    for k in range(_BPW // _L):
        t = ts_v[pl.ds(k * _L, _L)]
        j = (t * jnp.float32(_NB - 1)).astype(jnp.int32)
        j = jnp.minimum(jnp.maximum(j, 0), _NB - 2)
        g0 = plsc.load_gather(bkt_v, [j])
        g1 = plsc.load_gather(bkt_v, [j + 1])
        idx = j + (t >= g0).astype(jnp.int32) + (t >= g1).astype(jnp.int32)
        c, o = divmod(k * _L, _CHUNK)
        bidx_v[c, pl.ds(o, _L)] = idx
        plsc.store_scatter(norm_v, [lanes + (k * _L), zero], (t - mean) * inv)

    tcopies = [
        pltpu.async_copy(ttab_hbm.at[bidx_v.at[c]],
                         trows_v.at[pl.ds(c * _CHUNK, _CHUNK)], sem_g)
        for c in range(_NCH)
    ]

    for cp in ucopies:
        cp.wait()
    for cp in tcopies:
        cp.wait()

    half = _BPW // 2
    out_copies = []
    for h in range(2):
        def row_copy(r, carry):
            x = urows_v[r, pl.ds(0, 2 * _L)]
            a, b = plsc.unpack(x, format=plsc.PackFormat.INTERLEAVED,
                               preferred_element_type=jnp.float32)
            outbuf_v[r, pl.ds(0, _L)] = a
            outbuf_v[r, pl.ds(_L, _L)] = b
            outbuf_v[r, pl.ds(2 * _L, _L)] = trows_v[r, pl.ds(0, _L)]
            outbuf_v[r, pl.ds(3 * _L, _L)] = trows_v[r, pl.ds(_L, _L)]
            return carry
        lax.fori_loop(h * half, (h + 1) * half, row_copy, 0)
        lanes64 = lax.iota(jnp.int32, _L)
        col64 = jnp.full((_L,), 2 * _D, jnp.int32)
        for k in range(h * half // _L, (h + 1) * half // _L):
            plsc.store_scatter(outbuf_v,
                               [lanes64 + (k * _L), col64],
                               plsc.load_gather(norm_v,
                                                [lanes64 + (k * _L),
                                                 jnp.zeros((_L,), jnp.int32)]))
        out_copies.append(pltpu.async_copy(
            outbuf_v.at[pl.ds(h * half, half)],
            out_hbm.at[pl.ds(base + h * half, half)], sem_out))
    for cp in out_copies:
        cp.wait()


def kernel(user_id, timestamp, user_table, ts_table, buckets, mean, var):
    inv = jnp.float32(1.0) / jnp.sqrt(var.astype(jnp.float32) + 1e-7)
    cst = jnp.stack([jnp.full((_L,), mean, jnp.float32),
                     jnp.full((_L,), inv, jnp.float32)])
    uid3 = user_id.reshape(_NW, _NCH, _CHUNK)
    ut16 = user_table.astype(jnp.bfloat16)[:, _QPERM]

    mesh = plsc.VectorSubcoreMesh(core_axis_name="c", subcore_axis_name="s")
    f = pl.kernel(
        _tec_body,
        out_type=jax.ShapeDtypeStruct((_BATCH, 128), jnp.float32),
        mesh=mesh,
        compiler_params=pltpu.CompilerParams(needs_layout_passes=False,
                                             use_tc_tiling_on_sc=False),
        scratch_types=[
            pltpu.VMEM((_NCH, _CHUNK), jnp.int32),
            pltpu.VMEM((_BPW,), jnp.float32),
            pltpu.VMEM((_NCH, _CHUNK), jnp.int32),
            pltpu.VMEM((_BPW, _D), jnp.bfloat16),
            pltpu.VMEM((_BPW, _D), jnp.float32),
            pltpu.VMEM((_NB,), jnp.float32),
            pltpu.VMEM((2, _L), jnp.float32),
            pltpu.VMEM((_BPW, 1), jnp.float32),
            pltpu.VMEM((_BPW, 128), jnp.float32),
            pltpu.SemaphoreType.DMA,
            pltpu.SemaphoreType.DMA,
            pltpu.SemaphoreType.DMA,
        ],
    )
    out = f(uid3, timestamp, ut16, ts_table, buckets, cst)
    return out[:, :2 * _D + 1]


# speedup vs baseline: 1.5148x; 1.5148x over previous
import jax
import jax.numpy as jnp
from jax import lax
from jax.experimental import pallas as pl
from jax.experimental.pallas import tpu as pltpu
from jax.experimental.pallas import tpu_sc as plsc

_NC, _NS, _L = 2, 16, 16
_NW = _NC * _NS
_BATCH = 16384
_BPW = _BATCH // _NW
_CHUNK = 128
_NCH = _BPW // _CHUNK
_D = 32
_NB = 1000


def _tec_body(uid_hbm, ts_hbm, utab_hbm, ttab_hbm, bkt_hbm, cst_hbm,
              out_hbm,
              uid_v, ts_v, bidx_v, urows_v, trows_v, bkt_v, cst_v, norm_v,
              outbuf_v, sem_in, sems_u, sems_t, sem_out):
    wid = lax.axis_index("s") * _NC + lax.axis_index("c")
    base = wid * _BPW

    stage = [
        pltpu.async_copy(uid_hbm.at[wid], uid_v, sem_in),
        pltpu.async_copy(ts_hbm.at[pl.ds(base, _BPW)], ts_v, sem_in),
        pltpu.async_copy(bkt_hbm, bkt_v, sem_in),
        pltpu.async_copy(cst_hbm, cst_v, sem_in),
    ]
    for cp in stage:
        cp.wait()

    ucopies = [
        pltpu.async_copy(utab_hbm.at[uid_v.at[c]],
                         urows_v.at[pl.ds(c * _CHUNK, _CHUNK)], sems_u.at[c])
        for c in range(_NCH)
    ]

    mean = cst_v[0]
    inv = cst_v[1]
    lanes = lax.iota(jnp.int32, _L)
    zero = jnp.zeros((_L,), jnp.int32)
    for k in range(_BPW // _L):
        t = ts_v[pl.ds(k * _L, _L)]
        j = (t * jnp.float32(_NB - 1)).astype(jnp.int32)
        j = jnp.minimum(jnp.maximum(j, 0), _NB - 2)
        g0 = plsc.load_gather(bkt_v, [j])
        g1 = plsc.load_gather(bkt_v, [j + 1])
        idx = j + (t >= g0).astype(jnp.int32) + (t >= g1).astype(jnp.int32)
        c, o = divmod(k * _L, _CHUNK)
        bidx_v[c, pl.ds(o, _L)] = idx
        plsc.store_scatter(norm_v, [lanes + (k * _L), zero], (t - mean) * inv)

    tcopies = [
        pltpu.async_copy(ttab_hbm.at[bidx_v.at[c]],
                         trows_v.at[pl.ds(c * _CHUNK, _CHUNK)], sems_t.at[c])
        for c in range(_NCH)
    ]

    col64 = jnp.full((_L,), 2 * _D, jnp.int32)
    out_copies = []
    for c in range(_NCH):
        ucopies[c].wait()
        tcopies[c].wait()

        def row_copy(r, carry):
            outbuf_v[r, pl.ds(0, _L)] = urows_v[r, pl.ds(0, _L)]
            outbuf_v[r, pl.ds(_L, _L)] = urows_v[r, pl.ds(_L, _L)]
            outbuf_v[r, pl.ds(2 * _L, _L)] = trows_v[r, pl.ds(0, _L)]
            outbuf_v[r, pl.ds(3 * _L, _L)] = trows_v[r, pl.ds(_L, _L)]
            return carry
        lax.fori_loop(c * _CHUNK, (c + 1) * _CHUNK, row_copy, 0)
        for k in range(c * _CHUNK // _L, (c + 1) * _CHUNK // _L):
            plsc.store_scatter(outbuf_v,
                               [lanes + (k * _L), col64],
                               plsc.load_gather(norm_v,
                                                [lanes + (k * _L), zero]))
        out_copies.append(pltpu.async_copy(
            outbuf_v.at[pl.ds(c * _CHUNK, _CHUNK)],
            out_hbm.at[pl.ds(base + c * _CHUNK, _CHUNK)], sem_out))
    for cp in out_copies:
        cp.wait()


def kernel(user_id, timestamp, user_table, ts_table, buckets, mean, var):
    inv = jnp.float32(1.0) / jnp.sqrt(var.astype(jnp.float32) + 1e-7)
    cst = jnp.stack([jnp.full((_L,), mean, jnp.float32),
                     jnp.full((_L,), inv, jnp.float32)])
    uid3 = user_id.reshape(_NW, _NCH, _CHUNK)

    mesh = plsc.VectorSubcoreMesh(core_axis_name="c", subcore_axis_name="s")
    f = pl.kernel(
        _tec_body,
        out_type=jax.ShapeDtypeStruct((_BATCH, 128), jnp.float32),
        mesh=mesh,
        compiler_params=pltpu.CompilerParams(needs_layout_passes=False,
                                             use_tc_tiling_on_sc=False),
        scratch_types=[
            pltpu.VMEM((_NCH, _CHUNK), jnp.int32),
            pltpu.VMEM((_BPW,), jnp.float32),
            pltpu.VMEM((_NCH, _CHUNK), jnp.int32),
            pltpu.VMEM((_BPW, _D), jnp.float32),
            pltpu.VMEM((_BPW, _D), jnp.float32),
            pltpu.VMEM((_NB,), jnp.float32),
            pltpu.VMEM((2, _L), jnp.float32),
            pltpu.VMEM((_BPW, 1), jnp.float32),
            pltpu.VMEM((_BPW, 128), jnp.float32),
            pltpu.SemaphoreType.DMA,
            pltpu.SemaphoreType.DMA((_NCH,)),
            pltpu.SemaphoreType.DMA((_NCH,)),
            pltpu.SemaphoreType.DMA,
        ],
    )
    out = f(uid3, timestamp, user_table, ts_table, buckets, cst)
    return out[:, :2 * _D + 1]


# speedup vs baseline: 1.5226x; 1.0052x over previous
import jax
import jax.numpy as jnp
from jax import lax
from jax.experimental import pallas as pl
from jax.experimental.pallas import tpu as pltpu
from jax.experimental.pallas import tpu_sc as plsc

_NC, _NS, _L = 2, 16, 16
_NW = _NC * _NS
_BATCH = 16384
_BPW = _BATCH // _NW
_CHUNK = 128
_NCH = _BPW // _CHUNK
_D = 32
_NB = 1000


def _tec_body(uid_hbm, ts_hbm, utab_hbm, ttab_hbm, bkt_hbm, cst_hbm,
              out_hbm,
              uid_v, ts_v, bidx_v, urows_v, trows_v, bkt_v, cst_v,
              outbuf_v, sem_in, sems_u, sems_t, sem_out):
    wid = lax.axis_index("s") * _NC + lax.axis_index("c")
    base = wid * _BPW

    cp_uid = pltpu.async_copy(uid_hbm.at[wid], uid_v, sem_out)
    stage = [
        pltpu.async_copy(ts_hbm.at[pl.ds(base, _BPW)], ts_v, sem_in),
        pltpu.async_copy(bkt_hbm, bkt_v, sem_in),
        pltpu.async_copy(cst_hbm, cst_v, sem_in),
    ]
    cp_uid.wait()

    ucopies = [
        pltpu.async_copy(utab_hbm.at[uid_v.at[c]],
                         urows_v.at[pl.ds(c * _CHUNK, _CHUNK)], sems_u.at[c])
        for c in range(_NCH)
    ]
    for cp in stage:
        cp.wait()

    mean = cst_v[0]
    inv = cst_v[1]
    lanes = lax.iota(jnp.int32, _L)
    col64 = jnp.full((_L,), 2 * _D, jnp.int32)
    for k in range(_BPW // _L):
        t = ts_v[pl.ds(k * _L, _L)]
        j = (t * jnp.float32(_NB - 1)).astype(jnp.int32)
        j = jnp.minimum(jnp.maximum(j, 0), _NB - 2)
        g0 = plsc.load_gather(bkt_v, [j])
        g1 = plsc.load_gather(bkt_v, [j + 1])
        idx = j + (t >= g0).astype(jnp.int32) + (t >= g1).astype(jnp.int32)
        c, o = divmod(k * _L, _CHUNK)
        bidx_v[c, pl.ds(o, _L)] = idx
        plsc.store_scatter(outbuf_v, [lanes + (k * _L), col64],
                           (t - mean) * inv)

    tcopies = [
        pltpu.async_copy(ttab_hbm.at[bidx_v.at[c]],
                         trows_v.at[pl.ds(c * _CHUNK, _CHUNK)], sems_t.at[c])
        for c in range(_NCH)
    ]

    out_copies = []
    for c in range(_NCH):
        ucopies[c].wait()
        tcopies[c].wait()

        def row_copy(r, carry):
            outbuf_v[r, pl.ds(0, _L)] = urows_v[r, pl.ds(0, _L)]
            outbuf_v[r, pl.ds(_L, _L)] = urows_v[r, pl.ds(_L, _L)]
            outbuf_v[r, pl.ds(2 * _L, _L)] = trows_v[r, pl.ds(0, _L)]
            outbuf_v[r, pl.ds(3 * _L, _L)] = trows_v[r, pl.ds(_L, _L)]
            return carry
        lax.fori_loop(c * _CHUNK, (c + 1) * _CHUNK, row_copy, 0)
        out_copies.append(pltpu.async_copy(
            outbuf_v.at[pl.ds(c * _CHUNK, _CHUNK)],
            out_hbm.at[pl.ds(base + c * _CHUNK, _CHUNK)], sem_out))
    for cp in out_copies:
        cp.wait()


def kernel(user_id, timestamp, user_table, ts_table, buckets, mean, var):
    inv = jnp.float32(1.0) / jnp.sqrt(var.astype(jnp.float32) + 1e-7)
    cst = jnp.stack([jnp.full((_L,), mean, jnp.float32),
                     jnp.full((_L,), inv, jnp.float32)])
    uid3 = user_id.reshape(_NW, _NCH, _CHUNK)

    mesh = plsc.VectorSubcoreMesh(core_axis_name="c", subcore_axis_name="s")
    f = pl.kernel(
        _tec_body,
        out_type=jax.ShapeDtypeStruct((_BATCH, 128), jnp.float32),
        mesh=mesh,
        compiler_params=pltpu.CompilerParams(needs_layout_passes=False,
                                             use_tc_tiling_on_sc=False),
        scratch_types=[
            pltpu.VMEM((_NCH, _CHUNK), jnp.int32),
            pltpu.VMEM((_BPW,), jnp.float32),
            pltpu.VMEM((_NCH, _CHUNK), jnp.int32),
            pltpu.VMEM((_BPW, _D), jnp.float32),
            pltpu.VMEM((_BPW, _D), jnp.float32),
            pltpu.VMEM((_NB,), jnp.float32),
            pltpu.VMEM((2, _L), jnp.float32),
            pltpu.VMEM((_BPW, 128), jnp.float32),
            pltpu.SemaphoreType.DMA,
            pltpu.SemaphoreType.DMA((_NCH,)),
            pltpu.SemaphoreType.DMA((_NCH,)),
            pltpu.SemaphoreType.DMA,
        ],
    )
    out = f(uid3, timestamp, user_table, ts_table, buckets, cst)
    return out[:, :2 * _D + 1]
